# Initial kernel scaffold; baseline (speedup 1.0000x reference)
import jax
import jax.numpy as jnp
from jax.experimental import pallas as pl


def kernel(x, edge_index, edge_weight, W1, b1, W2, b2):
    raise NotImplementedError("write your pallas kernel here")


# speedup vs baseline: 8.9510x; 8.9510x over previous
import functools

import jax
import jax.numpy as jnp
from jax import lax
from jax.experimental import pallas as pl
from jax.experimental.pallas import tpu as pltpu
from jax.experimental.pallas import tpu_sc as plsc

N = 10000
D = 128
H = 64
K = 10
ALPHA = 0.1

NC = 2
NS = 16
NW = NC * NS
CH = 128
CPT = 82
TPW = CPT * CH
ET = NW * TPW
N_PAD = 10240
RPT = N_PAD // NS

_mesh = plsc.VectorSubcoreMesh(core_axis_name="c", subcore_axis_name="s")
_sc_params = pltpu.CompilerParams(use_tc_tiling_on_sc=False)


DW = 16


@functools.partial(
    pl.kernel,
    out_type=jax.ShapeDtypeStruct((NC, N_PAD, DW), jnp.float32),
    mesh=_mesh,
    scratch_types=[
        pltpu.VMEM_SHARED((N_PAD, DW), jnp.float32),
        pltpu.VMEM((CPT, CH), jnp.int32),
        pltpu.VMEM((CH, DW), jnp.float32),
    ],
    compiler_params=_sc_params,
)
def _deg_kernel(dstp_hbm, zcol_hbm, ones_hbm, out_hbm, deg_sh, di_v, ones_v):
    cid = lax.axis_index("c")
    sid = lax.axis_index("s")
    wid = sid * NC + cid
    pltpu.sync_copy(zcol_hbm, deg_sh.at[pl.ds(sid * RPT, RPT)])
    pltpu.sync_copy(ones_hbm, ones_v)
    pltpu.sync_copy(dstp_hbm.at[wid], di_v)
    plsc.subcore_barrier()

    def body(j, carry):
        pltpu.sync_copy(ones_v, deg_sh.at[di_v.at[j]], add=True)
        return carry

    lax.fori_loop(0, CPT, body, 0)
    plsc.subcore_barrier()
    pltpu.sync_copy(deg_sh.at[pl.ds(sid * RPT, RPT)],
                    out_hbm.at[cid, pl.ds(sid * RPT, RPT)])


@functools.partial(
    pl.kernel,
    out_type=jax.ShapeDtypeStruct((NC, N_PAD, H), jnp.float32),
    mesh=_mesh,
    scratch_types=[
        pltpu.VMEM_SHARED((N_PAD, H), jnp.float32),
        pltpu.VMEM((CPT, CH), jnp.int32),
        pltpu.VMEM((CPT, CH), jnp.int32),
        pltpu.VMEM((CH, H), jnp.float32),
        pltpu.SemaphoreType.DMA,
    ],
    compiler_params=_sc_params,
)
def _round_kernel(g_hbm, srcp_hbm, dstp_hbm, zrows_hbm, out_hbm,
                  p_sh, si_v, di_v, rows_v, sem):
    cid = lax.axis_index("c")
    sid = lax.axis_index("s")
    wid = sid * NC + cid
    pltpu.sync_copy(zrows_hbm, p_sh.at[pl.ds(sid * RPT, RPT)])
    pltpu.sync_copy(srcp_hbm.at[wid], si_v)
    pltpu.sync_copy(dstp_hbm.at[wid], di_v)
    plsc.subcore_barrier()

    def body(j, carry):
        pltpu.async_copy(g_hbm.at[si_v.at[j]], rows_v, sem).wait()
        pltpu.sync_copy(rows_v, p_sh.at[di_v.at[j]], add=True)
        return carry

    lax.fori_loop(0, CPT, body, 0)
    plsc.subcore_barrier()
    pltpu.sync_copy(p_sh.at[pl.ds(sid * RPT, RPT)],
                    out_hbm.at[cid, pl.ds(sid * RPT, RPT)])


def _prep_body(deg_ref, x_ref, w1_ref, y_ref, g_ref, dinv_ref):
    deg = deg_ref[0, :N, 0:1] + deg_ref[1, :N, 0:1]
    dinv = jnp.where(deg > 0.0, lax.rsqrt(deg), 0.0)
    y = jnp.dot(x_ref[...], w1_ref[...], preferred_element_type=jnp.float32)
    y_ref[...] = y
    g_ref[...] = y * dinv
    dinv_ref[...] = dinv


def _combine_body(p_ref, dinv_ref, acc_ref, acc_out_ref, g_ref):
    s = p_ref[0, :N, :] + p_ref[1, :N, :]
    dinv = dinv_ref[...]
    h = s * dinv
    acc_out_ref[...] = acc_ref[...] + h
    g_ref[...] = h * dinv


def _mlp_body(acc_ref, y_ref, b1_ref, w2_ref, b2_ref, out_ref):
    h = (1.0 - ALPHA) / K * acc_ref[...] + ALPHA * y_ref[...] + b1_ref[...]
    h = jnp.maximum(h, 0.0)
    out_ref[...] = (jnp.dot(h, w2_ref[...], preferred_element_type=jnp.float32)
                    + b2_ref[...])


def kernel(x, edge_index, edge_weight, W1, b1, W2, b2):
    src = edge_index[0].astype(jnp.int32)
    dst = edge_index[1].astype(jnp.int32)
    loop = jnp.arange(N, dtype=jnp.int32)
    src_full = jnp.concatenate([src, loop])
    dst_full = jnp.concatenate([dst, loop])
    e2 = src_full.shape[0]
    pad = ET - e2
    srcp = jnp.concatenate([src_full, jnp.zeros((pad,), jnp.int32)])
    dstp = jnp.concatenate([dst_full, jnp.full((pad,), N, jnp.int32)])
    srcp = srcp.reshape(NW, CPT, CH)
    dstp = dstp.reshape(NW, CPT, CH)

    zcol = jnp.zeros((RPT, DW), jnp.float32)
    ones_c = jnp.ones((CH, DW), jnp.float32)
    zrows = jnp.zeros((RPT, H), jnp.float32)

    deg_p = _deg_kernel(dstp, zcol, ones_c)

    y, g, dinv = pl.pallas_call(
        _prep_body,
        out_shape=[
            jax.ShapeDtypeStruct((N, H), jnp.float32),
            jax.ShapeDtypeStruct((N, H), jnp.float32),
            jax.ShapeDtypeStruct((N, 1), jnp.float32),
        ],
    )(deg_p, x, W1)

    acc = jnp.zeros((N, H), jnp.float32)
    for _ in range(K):
        p = _round_kernel(g, srcp, dstp, zrows)
        acc, g = pl.pallas_call(
            _combine_body,
            out_shape=[
                jax.ShapeDtypeStruct((N, H), jnp.float32),
                jax.ShapeDtypeStruct((N, H), jnp.float32),
            ],
        )(p, dinv, acc)

    out = pl.pallas_call(
        _mlp_body,
        out_shape=jax.ShapeDtypeStruct((N, W2.shape[1]), jnp.float32),
    )(acc, y, b1, W2, b2)
    return out


# speedup vs baseline: 9.7158x; 1.0854x over previous
import functools

import jax
import jax.numpy as jnp
from jax import lax
from jax.experimental import pallas as pl
from jax.experimental.pallas import tpu as pltpu
from jax.experimental.pallas import tpu_sc as plsc

N = 10000
D = 128
H = 64
K = 10
ALPHA = 0.1

NC = 2
NS = 16
NW = NC * NS
CH = 128
CPT = 82
TPW = CPT * CH
ET = NW * TPW
N_PAD = 10240
RPT = N_PAD // NS

_mesh = plsc.VectorSubcoreMesh(core_axis_name="c", subcore_axis_name="s")
_sc_params = pltpu.CompilerParams(use_tc_tiling_on_sc=False)


DW = 16


@functools.partial(
    pl.kernel,
    out_type=jax.ShapeDtypeStruct((NC, N_PAD, DW), jnp.float32),
    mesh=_mesh,
    scratch_types=[
        pltpu.VMEM_SHARED((N_PAD, DW), jnp.float32),
        pltpu.VMEM((CPT, CH), jnp.int32),
        pltpu.VMEM((CH, DW), jnp.float32),
    ],
    compiler_params=_sc_params,
)
def _deg_kernel(dstp_hbm, zcol_hbm, ones_hbm, out_hbm, deg_sh, di_v, ones_v):
    cid = lax.axis_index("c")
    sid = lax.axis_index("s")
    wid = sid * NC + cid
    pltpu.sync_copy(zcol_hbm, deg_sh.at[pl.ds(sid * RPT, RPT)])
    pltpu.sync_copy(ones_hbm, ones_v)
    pltpu.sync_copy(dstp_hbm.at[wid], di_v)
    plsc.subcore_barrier()

    def body(j, carry):
        pltpu.sync_copy(ones_v, deg_sh.at[di_v.at[j]], add=True)
        return carry

    lax.fori_loop(0, CPT, body, 0)
    plsc.subcore_barrier()
    pltpu.sync_copy(deg_sh.at[pl.ds(sid * RPT, RPT)],
                    out_hbm.at[cid, pl.ds(sid * RPT, RPT)])


@functools.partial(
    pl.kernel,
    out_type=jax.ShapeDtypeStruct((NC, N_PAD, H), jnp.float32),
    mesh=_mesh,
    scratch_types=[
        pltpu.VMEM_SHARED((N_PAD, H), jnp.float32),
        pltpu.VMEM((CPT, CH), jnp.int32),
        pltpu.VMEM((CPT, CH), jnp.int32),
        pltpu.VMEM((2, CH, H), jnp.float32),
        pltpu.SemaphoreType.DMA,
        pltpu.SemaphoreType.DMA,
    ],
    compiler_params=_sc_params,
)
def _round_kernel(g_hbm, srcp_hbm, dstp_hbm, zrows_hbm, out_hbm,
                  p_sh, si_v, di_v, rows_v, sem0, sem1):
    cid = lax.axis_index("c")
    sid = lax.axis_index("s")
    wid = sid * NC + cid
    pltpu.sync_copy(zrows_hbm, p_sh.at[pl.ds(sid * RPT, RPT)])
    pltpu.sync_copy(srcp_hbm.at[wid], si_v)
    pltpu.sync_copy(dstp_hbm.at[wid], di_v)
    plsc.subcore_barrier()

    pltpu.async_copy(g_hbm.at[si_v.at[0]], rows_v.at[0], sem0)

    def pair(p, carry):
        j0 = 2 * p
        j1 = j0 + 1
        j2 = jnp.minimum(j0 + 2, CPT - 1)
        pltpu.make_async_copy(g_hbm.at[si_v.at[j0]], rows_v.at[0], sem0).wait()
        pltpu.async_copy(g_hbm.at[si_v.at[j1]], rows_v.at[1], sem1)
        pltpu.sync_copy(rows_v.at[0], p_sh.at[di_v.at[j0]], add=True)
        pltpu.make_async_copy(g_hbm.at[si_v.at[j1]], rows_v.at[1], sem1).wait()
        pltpu.async_copy(g_hbm.at[si_v.at[j2]], rows_v.at[0], sem0)
        pltpu.sync_copy(rows_v.at[1], p_sh.at[di_v.at[j1]], add=True)
        return carry

    lax.fori_loop(0, CPT // 2, pair, 0)
    pltpu.make_async_copy(g_hbm.at[si_v.at[0]], rows_v.at[0], sem0).wait()
    plsc.subcore_barrier()
    pltpu.sync_copy(p_sh.at[pl.ds(sid * RPT, RPT)],
                    out_hbm.at[cid, pl.ds(sid * RPT, RPT)])


def _prep_body(deg_ref, x_ref, w1_ref, y_ref, g_ref, dinv_ref):
    deg = deg_ref[0, :N, 0:1] + deg_ref[1, :N, 0:1]
    dinv = jnp.where(deg > 0.0, lax.rsqrt(deg), 0.0)
    y = jnp.dot(x_ref[...], w1_ref[...], preferred_element_type=jnp.float32)
    y_ref[...] = y
    g_ref[...] = y * dinv
    dinv_ref[...] = dinv


def _combine_body(p_ref, dinv_ref, acc_ref, acc_out_ref, g_ref):
    s = p_ref[0, :N, :] + p_ref[1, :N, :]
    dinv = dinv_ref[...]
    h = s * dinv
    acc_out_ref[...] = acc_ref[...] + h
    g_ref[...] = h * dinv


def _mlp_body(acc_ref, y_ref, b1_ref, w2_ref, b2_ref, out_ref):
    h = (1.0 - ALPHA) / K * acc_ref[...] + ALPHA * y_ref[...] + b1_ref[...]
    h = jnp.maximum(h, 0.0)
    out_ref[...] = (jnp.dot(h, w2_ref[...], preferred_element_type=jnp.float32)
                    + b2_ref[...])


def kernel(x, edge_index, edge_weight, W1, b1, W2, b2):
    src = edge_index[0].astype(jnp.int32)
    dst = edge_index[1].astype(jnp.int32)
    loop = jnp.arange(N, dtype=jnp.int32)
    src_full = jnp.concatenate([src, loop])
    dst_full = jnp.concatenate([dst, loop])
    e2 = src_full.shape[0]
    pad = ET - e2
    srcp = jnp.concatenate([src_full, jnp.zeros((pad,), jnp.int32)])
    dstp = jnp.concatenate([dst_full, jnp.full((pad,), N, jnp.int32)])
    srcp = srcp.reshape(NW, CPT, CH)
    dstp = dstp.reshape(NW, CPT, CH)

    zcol = jnp.zeros((RPT, DW), jnp.float32)
    ones_c = jnp.ones((CH, DW), jnp.float32)
    zrows = jnp.zeros((RPT, H), jnp.float32)

    deg_p = _deg_kernel(dstp, zcol, ones_c)

    y, g, dinv = pl.pallas_call(
        _prep_body,
        out_shape=[
            jax.ShapeDtypeStruct((N, H), jnp.float32),
            jax.ShapeDtypeStruct((N, H), jnp.float32),
            jax.ShapeDtypeStruct((N, 1), jnp.float32),
        ],
    )(deg_p, x, W1)

    acc = jnp.zeros((N, H), jnp.float32)
    for _ in range(K):
        p = _round_kernel(g, srcp, dstp, zrows)
        acc, g = pl.pallas_call(
            _combine_body,
            out_shape=[
                jax.ShapeDtypeStruct((N, H), jnp.float32),
                jax.ShapeDtypeStruct((N, H), jnp.float32),
            ],
        )(p, dinv, acc)

    out = pl.pallas_call(
        _mlp_body,
        out_shape=jax.ShapeDtypeStruct((N, W2.shape[1]), jnp.float32),
    )(acc, y, b1, W2, b2)
    return out
